# baseline (speedup 1.0000x reference)
import functools

import jax
import jax.numpy as jnp
from jax import lax
from jax.experimental import pallas as pl
from jax.experimental.pallas import tpu as pltpu
from jax.experimental.pallas import tpu_sc as plsc

B_ = 16384
L_ = 50
E_ = 64
H_ = 128
V_ = 1000000
FLAT = B_ * L_


VB = 8192


def _build_body(tabT_ref, w_ref, b_ref, out_ref):
  acc = lax.dot_general(
      tabT_ref[...],
      w_ref[...],
      dimension_numbers=(((0,), (0,)), ((), ())),
      preferred_element_type=jnp.float32,
  )
  out_ref[...] = jnp.tanh(acc + b_ref[...])


def _build_t2(tableT, wT, b_row):
  return pl.pallas_call(
      _build_body,
      grid=(pl.cdiv(V_, VB),),
      in_specs=[
          pl.BlockSpec((E_, VB), lambda i: (0, i)),
          pl.BlockSpec((E_, H_), lambda i: (0, 0)),
          pl.BlockSpec((1, H_), lambda i: (0, 0)),
      ],
      out_specs=pl.BlockSpec((VB, H_), lambda i: (i, 0)),
      out_shape=jax.ShapeDtypeStruct((V_, H_), jnp.float32),
  )(tableT, wT, b_row)


NW = 32
CH = 128
PER_W = FLAT // NW
NCH = PER_W // CH
NBUF = 4


def _gather_kernel(idx_hbm, t2_hbm, out_hbm, idx_v, buf0, buf1, buf2, buf3,
                   g0, g1, g2, g3, w0, w1, w2, w3):
  wid = lax.axis_index("s") * 2 + lax.axis_index("c")
  row0 = wid * NCH
  pltpu.sync_copy(idx_hbm.at[pl.ds(row0, NCH)], idx_v)

  bufs = (buf0, buf1, buf2, buf3)
  gsems = (g0, g1, g2, g3)
  wsems = (w0, w1, w2, w3)

  def fire(si, b):
    pltpu.async_copy(t2_hbm.at[idx_v.at[si]], bufs[b], gsems[b])

  def drain_g(b):
    pltpu.make_async_copy(
        t2_hbm.at[idx_v.at[0]], bufs[b], gsems[b]).wait()

  def drain_w(b):
    pltpu.make_async_copy(
        bufs[b], out_hbm.at[pl.ds(0, CH)], wsems[b]).wait()

  def write(si, b):
    pltpu.async_copy(
        bufs[b], out_hbm.at[pl.ds((row0 + si) * CH, CH)], wsems[b])

  for b in range(NBUF - 1):
    fire(b, b)

  def step4(si2, carry):
    for b in range(NBUF):
      si = NBUF * si2 + b
      bf = (b + NBUF - 1) % NBUF

      @pl.when(si > 0)
      def _():
        drain_w(bf)

      @pl.when(si + NBUF - 1 < NCH)
      def _():
        fire(si + NBUF - 1, bf)

      drain_g(b)
      write(si, b)
    return carry

  lax.fori_loop(0, NCH // NBUF, step4, 0)
  drain_w((NCH - 1) % NBUF)


def _gather(idx2d, t2):
  mesh = plsc.VectorSubcoreMesh(core_axis_name="c", subcore_axis_name="s")
  k = functools.partial(
      pl.kernel,
      mesh=mesh,
      out_type=jax.ShapeDtypeStruct((FLAT, H_), jnp.float32),
      scratch_types=[
          pltpu.VMEM((NCH, CH), jnp.int32),
          pltpu.VMEM((CH, H_), jnp.float32),
          pltpu.VMEM((CH, H_), jnp.float32),
          pltpu.VMEM((CH, H_), jnp.float32),
          pltpu.VMEM((CH, H_), jnp.float32),
          pltpu.SemaphoreType.DMA,
          pltpu.SemaphoreType.DMA,
          pltpu.SemaphoreType.DMA,
          pltpu.SemaphoreType.DMA,
          pltpu.SemaphoreType.DMA,
          pltpu.SemaphoreType.DMA,
          pltpu.SemaphoreType.DMA,
          pltpu.SemaphoreType.DMA,
      ],
  )(_gather_kernel)
  return k(idx2d, t2)


def kernel(DA, table, W_eh, b_eh):
  tableT = table.T
  wT = W_eh.T
  t2 = _build_t2(tableT, wT, b_eh.reshape(1, H_))
  idx2d = DA.T.reshape(FLAT // CH, CH)
  out2d = _gather(idx2d, t2)
  return out2d.reshape(L_, B_, H_).transpose(1, 0, 2)


# speedup vs baseline: 1.0485x; 1.0485x over previous
import functools

import jax
import jax.numpy as jnp
from jax import lax
from jax.experimental import pallas as pl
from jax.experimental.pallas import tpu as pltpu
from jax.experimental.pallas import tpu_sc as plsc

B_ = 16384
L_ = 50
E_ = 64
H_ = 128
V_ = 1000000
FLAT = B_ * L_


VB = 16384


def _build_body(tabT_ref, w_ref, b_ref, out_ref):
  acc = lax.dot_general(
      tabT_ref[...],
      w_ref[...],
      dimension_numbers=(((0,), (0,)), ((), ())),
      preferred_element_type=jnp.float32,
  )
  out_ref[...] = jnp.tanh(acc + b_ref[...])


def _build_t2(tableT, wT, b_row):
  return pl.pallas_call(
      _build_body,
      grid=(pl.cdiv(V_, VB),),
      in_specs=[
          pl.BlockSpec((E_, VB), lambda i: (0, i)),
          pl.BlockSpec((E_, H_), lambda i: (0, 0)),
          pl.BlockSpec((1, H_), lambda i: (0, 0)),
      ],
      out_specs=pl.BlockSpec((VB, H_), lambda i: (i, 0)),
      out_shape=jax.ShapeDtypeStruct((V_, H_), jnp.float32),
  )(tableT, wT, b_row)


NW = 32
CH = 128
PER_W = FLAT // NW
NCH = PER_W // CH
NBUF = 4


def _gather_kernel(idx_hbm, t2_hbm, out_hbm, idx_v, buf0, buf1, buf2, buf3,
                   g0, g1, g2, g3, w0, w1, w2, w3):
  wid = lax.axis_index("s") * 2 + lax.axis_index("c")
  row0 = wid * NCH
  pltpu.sync_copy(idx_hbm.at[pl.ds(row0, NCH)], idx_v)

  bufs = (buf0, buf1, buf2, buf3)
  gsems = (g0, g1, g2, g3)
  wsems = (w0, w1, w2, w3)

  def fire(si, b):
    pltpu.async_copy(t2_hbm.at[idx_v.at[si]], bufs[b], gsems[b])

  def drain_g(b):
    pltpu.make_async_copy(
        t2_hbm.at[idx_v.at[0]], bufs[b], gsems[b]).wait()

  def drain_w(b):
    pltpu.make_async_copy(
        bufs[b], out_hbm.at[pl.ds(0, CH)], wsems[b]).wait()

  def write(si, b):
    pltpu.async_copy(
        bufs[b], out_hbm.at[pl.ds((row0 + si) * CH, CH)], wsems[b])

  for b in range(NBUF - 1):
    fire(b, b)

  def step4(si2, carry):
    for b in range(NBUF):
      si = NBUF * si2 + b
      bf = (b + NBUF - 1) % NBUF

      @pl.when(si > 0)
      def _():
        drain_w(bf)

      @pl.when(si + NBUF - 1 < NCH)
      def _():
        fire(si + NBUF - 1, bf)

      drain_g(b)
      write(si, b)
    return carry

  lax.fori_loop(0, NCH // NBUF, step4, 0)
  drain_w((NCH - 1) % NBUF)


def _gather(idx2d, t2):
  mesh = plsc.VectorSubcoreMesh(core_axis_name="c", subcore_axis_name="s")
  k = functools.partial(
      pl.kernel,
      mesh=mesh,
      out_type=jax.ShapeDtypeStruct((FLAT, H_), jnp.float32),
      scratch_types=[
          pltpu.VMEM((NCH, CH), jnp.int32),
          pltpu.VMEM((CH, H_), jnp.float32),
          pltpu.VMEM((CH, H_), jnp.float32),
          pltpu.VMEM((CH, H_), jnp.float32),
          pltpu.VMEM((CH, H_), jnp.float32),
          pltpu.SemaphoreType.DMA,
          pltpu.SemaphoreType.DMA,
          pltpu.SemaphoreType.DMA,
          pltpu.SemaphoreType.DMA,
          pltpu.SemaphoreType.DMA,
          pltpu.SemaphoreType.DMA,
          pltpu.SemaphoreType.DMA,
          pltpu.SemaphoreType.DMA,
      ],
  )(_gather_kernel)
  return k(idx2d, t2)


def kernel(DA, table, W_eh, b_eh):
  tableT = table.T
  wT = W_eh.T
  t2 = _build_t2(tableT, wT, b_eh.reshape(1, H_))
  idx2d = DA.T.reshape(FLAT // CH, CH)
  out2d = _gather(idx2d, t2)
  return out2d.reshape(L_, B_, H_).transpose(1, 0, 2)


# speedup vs baseline: 1.0607x; 1.0116x over previous
import functools

import jax
import jax.numpy as jnp
from jax import lax
from jax.experimental import pallas as pl
from jax.experimental.pallas import tpu as pltpu
from jax.experimental.pallas import tpu_sc as plsc

B_ = 16384
L_ = 50
E_ = 64
H_ = 128
V_ = 1000000
FLAT = B_ * L_


VB = 32768


def _build_body(tabT_ref, w_ref, b_ref, out_ref):
  acc = lax.dot_general(
      tabT_ref[...],
      w_ref[...],
      dimension_numbers=(((0,), (0,)), ((), ())),
      preferred_element_type=jnp.float32,
  )
  out_ref[...] = jnp.tanh(acc + b_ref[...])


def _build_t2(tableT, wT, b_row):
  return pl.pallas_call(
      _build_body,
      grid=(pl.cdiv(V_, VB),),
      in_specs=[
          pl.BlockSpec((E_, VB), lambda i: (0, i)),
          pl.BlockSpec((E_, H_), lambda i: (0, 0)),
          pl.BlockSpec((1, H_), lambda i: (0, 0)),
      ],
      out_specs=pl.BlockSpec((VB, H_), lambda i: (i, 0)),
      out_shape=jax.ShapeDtypeStruct((V_, H_), jnp.float32),
  )(tableT, wT, b_row)


NW = 32
CH = 128
PER_W = FLAT // NW
NCH = PER_W // CH
NBUF = 4


def _gather_kernel(idx_hbm, t2_hbm, out_hbm, idx_v, buf0, buf1, buf2, buf3,
                   g0, g1, g2, g3, w0, w1, w2, w3):
  wid = lax.axis_index("s") * 2 + lax.axis_index("c")
  row0 = wid * NCH
  pltpu.sync_copy(idx_hbm.at[pl.ds(row0, NCH)], idx_v)

  bufs = (buf0, buf1, buf2, buf3)
  gsems = (g0, g1, g2, g3)
  wsems = (w0, w1, w2, w3)

  def fire(si, b):
    pltpu.async_copy(t2_hbm.at[idx_v.at[si]], bufs[b], gsems[b])

  def drain_g(b):
    pltpu.make_async_copy(
        t2_hbm.at[idx_v.at[0]], bufs[b], gsems[b]).wait()

  def drain_w(b):
    pltpu.make_async_copy(
        bufs[b], out_hbm.at[pl.ds(0, CH)], wsems[b]).wait()

  def write(si, b):
    pltpu.async_copy(
        bufs[b], out_hbm.at[pl.ds((row0 + si) * CH, CH)], wsems[b])

  for b in range(NBUF - 1):
    fire(b, b)

  def step4(si2, carry):
    for b in range(NBUF):
      si = NBUF * si2 + b
      bf = (b + NBUF - 1) % NBUF

      @pl.when(si > 0)
      def _():
        drain_w(bf)

      @pl.when(si + NBUF - 1 < NCH)
      def _():
        fire(si + NBUF - 1, bf)

      drain_g(b)
      write(si, b)
    return carry

  lax.fori_loop(0, NCH // NBUF, step4, 0)
  drain_w((NCH - 1) % NBUF)


def _gather(idx2d, t2):
  mesh = plsc.VectorSubcoreMesh(core_axis_name="c", subcore_axis_name="s")
  k = functools.partial(
      pl.kernel,
      mesh=mesh,
      out_type=jax.ShapeDtypeStruct((FLAT, H_), jnp.float32),
      scratch_types=[
          pltpu.VMEM((NCH, CH), jnp.int32),
          pltpu.VMEM((CH, H_), jnp.float32),
          pltpu.VMEM((CH, H_), jnp.float32),
          pltpu.VMEM((CH, H_), jnp.float32),
          pltpu.VMEM((CH, H_), jnp.float32),
          pltpu.SemaphoreType.DMA,
          pltpu.SemaphoreType.DMA,
          pltpu.SemaphoreType.DMA,
          pltpu.SemaphoreType.DMA,
          pltpu.SemaphoreType.DMA,
          pltpu.SemaphoreType.DMA,
          pltpu.SemaphoreType.DMA,
          pltpu.SemaphoreType.DMA,
      ],
  )(_gather_kernel)
  return k(idx2d, t2)


def kernel(DA, table, W_eh, b_eh):
  tableT = table.T
  wT = W_eh.T
  t2 = _build_t2(tableT, wT, b_eh.reshape(1, H_))
  idx2d = DA.T.reshape(FLAT // CH, CH)
  out2d = _gather(idx2d, t2)
  return out2d.reshape(L_, B_, H_).transpose(1, 0, 2)


# speedup vs baseline: 1.0608x; 1.0001x over previous
import functools

import jax
import jax.numpy as jnp
from jax import lax
from jax.experimental import pallas as pl
from jax.experimental.pallas import tpu as pltpu
from jax.experimental.pallas import tpu_sc as plsc

B_ = 16384
L_ = 50
E_ = 64
H_ = 128
V_ = 1000000
FLAT = B_ * L_


VB = 32768


def _build_body(tabT_ref, w_ref, b_ref, out_ref):
  acc = lax.dot_general(
      tabT_ref[...],
      w_ref[...],
      dimension_numbers=(((0,), (0,)), ((), ())),
      preferred_element_type=jnp.float32,
  )
  out_ref[...] = jnp.tanh(acc + b_ref[...])


def _build_t2(tableT, wT, b_row):
  return pl.pallas_call(
      _build_body,
      grid=(pl.cdiv(V_, VB),),
      in_specs=[
          pl.BlockSpec((E_, VB), lambda i: (0, i)),
          pl.BlockSpec((E_, H_), lambda i: (0, 0)),
          pl.BlockSpec((1, H_), lambda i: (0, 0)),
      ],
      out_specs=pl.BlockSpec((VB, H_), lambda i: (i, 0)),
      out_shape=jax.ShapeDtypeStruct((V_, H_), jnp.float32),
  )(tableT, wT, b_row)


NW = 32
CH = 128
PER_W = FLAT // NW
NCH = PER_W // CH
NBUF = 5


def _gather_kernel(idx_hbm, t2_hbm, out_hbm, idx_v,
                   buf0, buf1, buf2, buf3, buf4,
                   g0, g1, g2, g3, g4, w0, w1, w2, w3, w4):
  wid = lax.axis_index("s") * 2 + lax.axis_index("c")
  row0 = wid * NCH
  pltpu.sync_copy(idx_hbm.at[pl.ds(row0, NCH)], idx_v)

  bufs = (buf0, buf1, buf2, buf3, buf4)
  gsems = (g0, g1, g2, g3, g4)
  wsems = (w0, w1, w2, w3, w4)

  def fire(si, b):
    pltpu.async_copy(t2_hbm.at[idx_v.at[si]], bufs[b], gsems[b])

  def drain_g(b):
    pltpu.make_async_copy(
        t2_hbm.at[idx_v.at[0]], bufs[b], gsems[b]).wait()

  def drain_w(b):
    pltpu.make_async_copy(
        bufs[b], out_hbm.at[pl.ds(0, CH)], wsems[b]).wait()

  def write(si, b):
    pltpu.async_copy(
        bufs[b], out_hbm.at[pl.ds((row0 + si) * CH, CH)], wsems[b])

  for b in range(NBUF - 1):
    fire(b, b)

  def step4(si2, carry):
    for b in range(NBUF):
      si = NBUF * si2 + b
      bf = (b + NBUF - 1) % NBUF

      @pl.when(si > 0)
      def _():
        drain_w(bf)

      @pl.when(si + NBUF - 1 < NCH)
      def _():
        fire(si + NBUF - 1, bf)

      drain_g(b)
      write(si, b)
    return carry

  lax.fori_loop(0, NCH // NBUF, step4, 0)
  drain_w((NCH - 1) % NBUF)


def _gather(idx2d, t2):
  mesh = plsc.VectorSubcoreMesh(core_axis_name="c", subcore_axis_name="s")
  k = functools.partial(
      pl.kernel,
      mesh=mesh,
      out_type=jax.ShapeDtypeStruct((FLAT, H_), jnp.float32),
      scratch_types=[
          pltpu.VMEM((NCH, CH), jnp.int32),
          pltpu.VMEM((CH, H_), jnp.float32),
          pltpu.VMEM((CH, H_), jnp.float32),
          pltpu.VMEM((CH, H_), jnp.float32),
          pltpu.VMEM((CH, H_), jnp.float32),
          pltpu.VMEM((CH, H_), jnp.float32),
          pltpu.SemaphoreType.DMA,
          pltpu.SemaphoreType.DMA,
          pltpu.SemaphoreType.DMA,
          pltpu.SemaphoreType.DMA,
          pltpu.SemaphoreType.DMA,
          pltpu.SemaphoreType.DMA,
          pltpu.SemaphoreType.DMA,
          pltpu.SemaphoreType.DMA,
          pltpu.SemaphoreType.DMA,
          pltpu.SemaphoreType.DMA,
      ],
  )(_gather_kernel)
  return k(idx2d, t2)


def kernel(DA, table, W_eh, b_eh):
  tableT = table.T
  wT = W_eh.T
  t2 = _build_t2(tableT, wT, b_eh.reshape(1, H_))
  idx2d = DA.T.reshape(FLAT // CH, CH)
  out2d = _gather(idx2d, t2)
  return out2d.reshape(L_, B_, H_).transpose(1, 0, 2)
